# Initial kernel scaffold; baseline (speedup 1.0000x reference)
import jax
import jax.numpy as jnp
from jax.experimental import pallas as pl


def kernel(aa_indices, seq_lengths, aa_table, pos_table):
    raise NotImplementedError("write your pallas kernel here")


# speedup vs baseline: 4.9966x; 4.9966x over previous
import functools

import jax
import jax.numpy as jnp
from jax import lax
from jax.experimental import pallas as pl
from jax.experimental.pallas import tpu as pltpu
from jax.experimental.pallas import tpu_sc as plsc

_NUM_AA = 20
_L = 200
_D = 128
_NC, _NS = 2, 16
_NW = _NC * _NS
_LANES = 16
_CHUNK = 128


def _prep_body(aa_ref, pos_ref, len_ref, c_ref, mask_ref):
    c_ref[...] = pos_ref[...][:, None, :] + aa_ref[...][None, :, :]
    pos_iota = lax.broadcasted_iota(jnp.int32, mask_ref.shape, 1)
    mask_ref[...] = pos_iota >= len_ref[...][:, None]


def kernel(aa_indices, seq_lengths, aa_table, pos_table):
    B, Lc = aa_indices.shape
    idx_flat = aa_indices.astype(jnp.int32).reshape(-1)
    seq_lengths = seq_lengths.astype(jnp.int32)

    c, mask = pl.pallas_call(
        _prep_body,
        out_shape=(
            jax.ShapeDtypeStruct((_L, _NUM_AA, _D), jnp.float32),
            jax.ShapeDtypeStruct((B, Lc), jnp.bool_),
        ),
    )(aa_table, pos_table, seq_lengths)
    c2 = c.reshape(_L * _NUM_AA, _D)

    R = B * Lc
    rows_per_w = R // _NW
    n_chunks = rows_per_w // _CHUNK

    mesh = plsc.VectorSubcoreMesh(core_axis_name="c", subcore_axis_name="s")

    @functools.partial(
        pl.kernel,
        out_type=jax.ShapeDtypeStruct((R, _D), jnp.float32),
        mesh=mesh,
        scratch_types=[
            pltpu.VMEM((_CHUNK,), jnp.int32),
            pltpu.VMEM((_CHUNK,), jnp.int32),
            pltpu.VMEM((_CHUNK, _D), jnp.float32),
            pltpu.SemaphoreType.DMA,
        ],
    )
    def sc_gather(idx_hbm, c_hbm, out_hbm, idx_v, flat_v, buf_v, sem):
        wid = lax.axis_index("s") * _NC + lax.axis_index("c")
        base = wid * rows_per_w

        def body(i, carry):
            r0 = base + i * _CHUNK
            pltpu.sync_copy(idx_hbm.at[pl.ds(r0, _CHUNK)], idx_v)
            for g in range(_CHUNK // _LANES):
                rvec = r0 + g * _LANES + lax.iota(jnp.int32, _LANES)
                p = lax.rem(rvec, _L)
                flat_v[pl.ds(g * _LANES, _LANES)] = (
                    idx_v[pl.ds(g * _LANES, _LANES)] + p * _NUM_AA)
            pltpu.async_copy(c_hbm.at[flat_v], buf_v, sem).wait()
            pltpu.sync_copy(buf_v, out_hbm.at[pl.ds(r0, _CHUNK)])
            return carry

        lax.fori_loop(0, n_chunks, body, 0)

    out = sc_gather(idx_flat, c2)
    return out.reshape(B, Lc, _D), mask


# speedup vs baseline: 8.1420x; 1.6295x over previous
import functools

import jax
import jax.numpy as jnp
from jax import lax
from jax.experimental import pallas as pl
from jax.experimental.pallas import tpu as pltpu
from jax.experimental.pallas import tpu_sc as plsc

_NUM_AA = 20
_L = 200
_D = 128
_NC, _NS = 2, 16
_NW = _NC * _NS
_LANES = 16
_CHUNK = 128
_NBUF = 4


def _prep_body(aa_ref, pos_ref, len_ref, c_ref, mask_ref):
    c_ref[...] = pos_ref[...][:, None, :] + aa_ref[...][None, :, :]
    pos_iota = lax.broadcasted_iota(jnp.int32, mask_ref.shape, 1)
    mask_ref[...] = pos_iota >= len_ref[...][:, None]


def kernel(aa_indices, seq_lengths, aa_table, pos_table):
    B, Lc = aa_indices.shape
    seq_lengths = seq_lengths.astype(jnp.int32)

    c, mask = pl.pallas_call(
        _prep_body,
        out_shape=(
            jax.ShapeDtypeStruct((_L, _NUM_AA, _D), jnp.float32),
            jax.ShapeDtypeStruct((B, Lc), jnp.bool_),
        ),
    )(aa_table, pos_table, seq_lengths)
    c2 = c.reshape(_L * _NUM_AA, _D)

    R = B * Lc
    rows_per_w = R // _NW
    n_chunks = rows_per_w // _CHUNK
    idx3 = aa_indices.astype(jnp.int32).reshape(_NW, n_chunks, _CHUNK)

    mesh = plsc.VectorSubcoreMesh(core_axis_name="c", subcore_axis_name="s")

    @functools.partial(
        pl.kernel,
        out_type=jax.ShapeDtypeStruct((R, _D), jnp.float32),
        mesh=mesh,
        scratch_types=[
            pltpu.VMEM((n_chunks, _CHUNK), jnp.int32),
            pltpu.VMEM((_NBUF, _CHUNK, _D), jnp.float32),
            [pltpu.SemaphoreType.DMA] * _NBUF,
            [pltpu.SemaphoreType.DMA] * _NBUF,
        ],
    )
    def sc_gather(idx_hbm, c_hbm, out_hbm, flat_v, bufs, gsems, wsems):
        wid = lax.axis_index("s") * _NC + lax.axis_index("c")
        base = wid * rows_per_w

        pltpu.sync_copy(idx_hbm.at[wid], flat_v)

        def flat_body(k, carry):
            for g in range(_CHUNK // _LANES):
                off = k * _CHUNK + g * _LANES
                p = lax.rem(off + lax.iota(jnp.int32, _LANES), _L)
                sl = pl.ds(g * _LANES, _LANES)
                flat_v[k, sl] = flat_v[k, sl] + p * _NUM_AA
            return carry

        lax.fori_loop(0, n_chunks, flat_body, 0)

        def start_gather(k, b):
            pltpu.async_copy(c_hbm.at[flat_v.at[k]], bufs.at[b], gsems[b])

        def wait_gather(b):
            pltpu.make_async_copy(
                c_hbm.at[flat_v.at[0]], bufs.at[b], gsems[b]).wait()

        def start_write(k, b):
            pltpu.async_copy(
                bufs.at[b], out_hbm.at[pl.ds(base + k * _CHUNK, _CHUNK)],
                wsems[b])

        def wait_write(b):
            pltpu.make_async_copy(
                bufs.at[b], out_hbm.at[pl.ds(base, _CHUNK)], wsems[b]).wait()

        start_gather(0, 0)
        start_gather(1, 1)
        for c in (0, 1):
            wait_gather(c % _NBUF)
            start_write(c, c % _NBUF)
            start_gather(c + 2, (c + 2) % _NBUF)

        def main_body(j, carry):
            for b in range(_NBUF):
                c = 2 + j * _NBUF + b
                bc = (2 + b) % _NBUF
                bn = (bc + 2) % _NBUF
                wait_gather(bc)
                start_write(c, bc)
                wait_write(bn)
                start_gather(c + 2, bn)
            return carry

        lax.fori_loop(0, (n_chunks - 4) // _NBUF, main_body, 0)

        for c in (n_chunks - 2, n_chunks - 1):
            bc = c % _NBUF
            wait_gather(bc)
            start_write(c, bc)
            wait_write((bc + 2) % _NBUF)
        for c in (n_chunks - 2, n_chunks - 1):
            wait_write(c % _NBUF)

    out = sc_gather(idx3, c2)
    return out.reshape(B, Lc, _D), mask


# speedup vs baseline: 14.8541x; 1.8244x over previous
import functools

import jax
import jax.numpy as jnp
from jax import lax
from jax.experimental import pallas as pl
from jax.experimental.pallas import tpu as pltpu
from jax.experimental.pallas import tpu_sc as plsc

_NUM_AA = 20
_L = 200
_D = 128
_NC, _NS = 2, 16
_NW = _NC * _NS
_LANES = 16
_CHUNK = 128
_NBUF = 4


def _prep_body(aa_ref, pos_ref, len_ref, c_ref, mask_ref):
    c_ref[...] = pos_ref[...][:, None, :] + aa_ref[...][None, :, :]
    pos_iota = lax.broadcasted_iota(jnp.int32, mask_ref.shape, 1)
    mask_ref[...] = pos_iota >= len_ref[...][:, None]


def kernel(aa_indices, seq_lengths, aa_table, pos_table):
    B, Lc = aa_indices.shape
    seq_lengths = seq_lengths.astype(jnp.int32)

    c, mask = pl.pallas_call(
        _prep_body,
        out_shape=(
            jax.ShapeDtypeStruct((_L, _NUM_AA, _D), jnp.float32),
            jax.ShapeDtypeStruct((B, Lc), jnp.bool_),
        ),
    )(aa_table, pos_table, seq_lengths)
    c2 = c.reshape(_L * _NUM_AA, _D)

    R = B * Lc
    rows_per_w = R // _NW
    n_chunks = rows_per_w // _CHUNK
    idx3 = aa_indices.astype(jnp.int32).reshape(_NW, n_chunks, _CHUNK)

    mesh = plsc.VectorSubcoreMesh(core_axis_name="c", subcore_axis_name="s")

    @functools.partial(
        pl.kernel,
        out_type=jax.ShapeDtypeStruct((R, _D), jnp.float32),
        mesh=mesh,
        scratch_types=[
            pltpu.VMEM((n_chunks, _CHUNK), jnp.int32),
            pltpu.VMEM((_NBUF, _CHUNK, _D), jnp.float32),
            pltpu.VMEM_SHARED((_L * _NUM_AA, _D), jnp.float32),
            [pltpu.SemaphoreType.DMA] * _NBUF,
            [pltpu.SemaphoreType.DMA] * _NBUF,
        ],
    )
    def sc_gather(idx_hbm, c_hbm, out_hbm, flat_v, bufs, c_sh, gsems, wsems):
        wid = lax.axis_index("s") * _NC + lax.axis_index("c")
        base = wid * rows_per_w

        @pl.when(lax.axis_index("s") == 0)
        def _load_table():
            pltpu.sync_copy(c_hbm, c_sh)

        plsc.subcore_barrier()

        pltpu.sync_copy(idx_hbm.at[wid], flat_v)

        def flat_body(k, carry):
            for g in range(_CHUNK // _LANES):
                off = k * _CHUNK + g * _LANES
                p = lax.rem(off + lax.iota(jnp.int32, _LANES), _L)
                sl = pl.ds(g * _LANES, _LANES)
                flat_v[k, sl] = flat_v[k, sl] + p * _NUM_AA
            return carry

        lax.fori_loop(0, n_chunks, flat_body, 0)

        def start_gather(k, b):
            pltpu.async_copy(c_sh.at[flat_v.at[k]], bufs.at[b], gsems[b])

        def wait_gather(b):
            pltpu.make_async_copy(
                c_sh.at[flat_v.at[0]], bufs.at[b], gsems[b]).wait()

        def start_write(k, b):
            pltpu.async_copy(
                bufs.at[b], out_hbm.at[pl.ds(base + k * _CHUNK, _CHUNK)],
                wsems[b])

        def wait_write(b):
            pltpu.make_async_copy(
                bufs.at[b], out_hbm.at[pl.ds(base, _CHUNK)], wsems[b]).wait()

        start_gather(0, 0)
        start_gather(1, 1)
        for c in (0, 1):
            wait_gather(c % _NBUF)
            start_write(c, c % _NBUF)
            start_gather(c + 2, (c + 2) % _NBUF)

        def main_body(j, carry):
            for b in range(_NBUF):
                c = 2 + j * _NBUF + b
                bc = (2 + b) % _NBUF
                bn = (bc + 2) % _NBUF
                wait_gather(bc)
                start_write(c, bc)
                wait_write(bn)
                start_gather(c + 2, bn)
            return carry

        lax.fori_loop(0, (n_chunks - 4) // _NBUF, main_body, 0)

        for c in (n_chunks - 2, n_chunks - 1):
            bc = c % _NBUF
            wait_gather(bc)
            start_write(c, bc)
            wait_write((bc + 2) % _NBUF)
        for c in (n_chunks - 2, n_chunks - 1):
            wait_write(c % _NBUF)

    out = sc_gather(idx3, c2)
    return out.reshape(B, Lc, _D), mask


# speedup vs baseline: 14.9982x; 1.0097x over previous
import functools

import jax
import jax.numpy as jnp
from jax import lax
from jax.experimental import pallas as pl
from jax.experimental.pallas import tpu as pltpu
from jax.experimental.pallas import tpu_sc as plsc

_NUM_AA = 20
_L = 200
_D = 128
_NC, _NS = 2, 16
_NW = _NC * _NS
_LANES = 16
_CHUNK = 128
_NBUF = 4


def _prep_body(aa_ref, pos_ref, len_ref, c_ref, mask_ref):
    c_ref[...] = pos_ref[...][:, None, :] + aa_ref[...][None, :, :]
    pos_iota = lax.broadcasted_iota(jnp.int32, mask_ref.shape, 1)
    mask_ref[...] = pos_iota >= len_ref[...][:, None]


def kernel(aa_indices, seq_lengths, aa_table, pos_table):
    B, Lc = aa_indices.shape
    seq_lengths = seq_lengths.astype(jnp.int32)

    c, mask = pl.pallas_call(
        _prep_body,
        out_shape=(
            jax.ShapeDtypeStruct((_L, _NUM_AA, _D), jnp.float32),
            jax.ShapeDtypeStruct((B, Lc), jnp.bool_),
        ),
    )(aa_table, pos_table, seq_lengths)
    c2 = c.reshape(_L * _NUM_AA, _D)

    R = B * Lc
    rows_per_w = R // _NW
    n_chunks = rows_per_w // _CHUNK
    idx3 = aa_indices.astype(jnp.int32).reshape(_NW, n_chunks, _CHUNK)

    mesh = plsc.VectorSubcoreMesh(core_axis_name="c", subcore_axis_name="s")

    @functools.partial(
        pl.kernel,
        out_type=jax.ShapeDtypeStruct((R, _D), jnp.float32),
        mesh=mesh,
        scratch_types=[
            pltpu.VMEM((n_chunks, _CHUNK), jnp.int32),
            pltpu.VMEM((_NBUF, _CHUNK, _D), jnp.float32),
            pltpu.VMEM_SHARED((_L * _NUM_AA, _D), jnp.float32),
            [pltpu.SemaphoreType.DMA] * _NBUF,
            [pltpu.SemaphoreType.DMA] * _NBUF,
        ],
    )
    def sc_gather(idx_hbm, c_hbm, out_hbm, flat_v, bufs, c_sh, gsems, wsems):
        wid = lax.axis_index("s") * _NC + lax.axis_index("c")
        base = wid * rows_per_w

        @pl.when(lax.axis_index("s") == 0)
        def _load_table():
            pltpu.sync_copy(c_hbm, c_sh)

        plsc.subcore_barrier()

        pltpu.sync_copy(idx_hbm.at[wid], flat_v)

        def compute_flat(k):
            m = lax.rem(k * _CHUNK, _L)
            for g in range(_CHUNK // _LANES):
                p = m + g * _LANES + lax.iota(jnp.int32, _LANES)
                p = jnp.where(p >= _L, p - _L, p)
                sl = pl.ds(g * _LANES, _LANES)
                flat_v[k, sl] = flat_v[k, sl] + p * _NUM_AA

        def start_gather(k, b):
            pltpu.async_copy(c_sh.at[flat_v.at[k]], bufs.at[b], gsems[b])

        def wait_gather(b):
            pltpu.make_async_copy(
                c_sh.at[flat_v.at[0]], bufs.at[b], gsems[b]).wait()

        def start_write(k, b):
            pltpu.async_copy(
                bufs.at[b], out_hbm.at[pl.ds(base + k * _CHUNK, _CHUNK)],
                wsems[b])

        def wait_write(b):
            pltpu.make_async_copy(
                bufs.at[b], out_hbm.at[pl.ds(base, _CHUNK)], wsems[b]).wait()

        compute_flat(0)
        compute_flat(1)
        start_gather(0, 0)
        start_gather(1, 1)
        for c in (0, 1):
            compute_flat(c + 2)
            wait_gather(c % _NBUF)
            start_write(c, c % _NBUF)
            start_gather(c + 2, (c + 2) % _NBUF)

        def main_body(j, carry):
            for b in range(_NBUF):
                c = 2 + j * _NBUF + b
                bc = (2 + b) % _NBUF
                bn = (bc + 2) % _NBUF
                compute_flat(c + 2)
                wait_gather(bc)
                start_write(c, bc)
                wait_write(bn)
                start_gather(c + 2, bn)
            return carry

        lax.fori_loop(0, (n_chunks - 4) // _NBUF, main_body, 0)

        for c in (n_chunks - 2, n_chunks - 1):
            bc = c % _NBUF
            wait_gather(bc)
            start_write(c, bc)
            wait_write((bc + 2) % _NBUF)
        for c in (n_chunks - 2, n_chunks - 1):
            wait_write(c % _NBUF)

    out = sc_gather(idx3, c2)
    return out.reshape(B, Lc, _D), mask
